# baseline (speedup 1.0000x reference)
import functools

import jax
import jax.numpy as jnp
from jax import lax
from jax.experimental import pallas as pl
from jax.experimental.pallas import tpu as pltpu
from jax.experimental.pallas import tpu_sc as plsc

D = 64
CHUNK = 128
NBUF = 8
NC = 2
NS = 16
NW = NC * NS


@functools.lru_cache(maxsize=None)
def _build(total_rows: int, chunks_per_w: int):
    ngrp = chunks_per_w // NBUF
    assert chunks_per_w % NBUF == 0

    mesh = plsc.VectorSubcoreMesh(core_axis_name="c", subcore_axis_name="s")

    @functools.partial(
        pl.kernel,
        mesh=mesh,
        out_type=jax.ShapeDtypeStruct((total_rows, D), jnp.float32),
        compiler_params=pltpu.CompilerParams(use_tc_tiling_on_sc=False),
        scratch_types=(
            [
                pltpu.VMEM((chunks_per_w, CHUNK), jnp.int32),
                pltpu.VMEM((NBUF, CHUNK, D), jnp.float32),
            ]
            + [pltpu.SemaphoreType.DMA] * (2 * NBUF)
        ),
    )
    def run(idx_hbm, table_hbm, out_hbm, idx_v, rows_v, *sems):
        sem_g = sems[:NBUF]
        sem_s = sems[NBUF:]
        wid = lax.axis_index("s") * NC + lax.axis_index("c")
        pltpu.sync_copy(idx_hbm.at[wid], idx_v)
        base_row = wid * (chunks_per_w * CHUNK)

        def group(g, carry):
            for b in range(NBUF):

                @pl.when(g > 0)
                def _():
                    pltpu.make_async_copy(
                        rows_v.at[b], out_hbm.at[pl.ds(0, CHUNK)], sem_s[b]
                    ).wait()

            gathers = []
            for b in range(NBUF):
                j = g * NBUF + b
                gathers.append(
                    pltpu.async_copy(
                        table_hbm.at[idx_v.at[j]], rows_v.at[b], sem_g[b]
                    )
                )
            for b in range(NBUF):
                j = g * NBUF + b
                gathers[b].wait()
                pltpu.async_copy(
                    rows_v.at[b],
                    out_hbm.at[pl.ds(base_row + j * CHUNK, CHUNK)],
                    sem_s[b],
                )
            return carry

        lax.fori_loop(0, ngrp, group, 0)
        for b in range(NBUF):
            pltpu.make_async_copy(
                rows_v.at[b], out_hbm.at[pl.ds(0, CHUNK)], sem_s[b]
            ).wait()

    return run


def kernel(x, table):
    B, S = x.shape
    total = B * S
    chunks_per_w = total // (NW * CHUNK)
    idx = x.reshape(NW, chunks_per_w, CHUNK).astype(jnp.int32)
    out = _build(total, chunks_per_w)(idx, table)
    return out.reshape(B, S, D)
